# speedup vs baseline: 4.8074x; 4.8074x over previous
import functools

import jax
import jax.numpy as jnp
from jax import lax
from jax.experimental import pallas as pl
from jax.experimental.pallas import tpu as pltpu
from jax.experimental.pallas import tpu_sc as plsc

N = 10000
E = 320000
D = 128

NC = 2
NS = 16
NW = NC * NS
B = 128
NBLK = -(-E // (NW * B))
E_PAD = NW * B * NBLK
N_PAD = 10240
RPT = N_PAD // NS


def _sc_segment_sum(x, src3, dst3):
    mesh = plsc.VectorSubcoreMesh(core_axis_name="c", subcore_axis_name="s")

    @functools.partial(
        pl.kernel,
        out_type=jax.ShapeDtypeStruct((NC * N_PAD, D), jnp.float32),
        mesh=mesh,
        scratch_types=dict(
            acc=pltpu.VMEM_SHARED((N_PAD, D), jnp.float32),
            src_v=pltpu.VMEM((NBLK, B), jnp.int32),
            dst_v=pltpu.VMEM((NBLK, B), jnp.int32),
            rows_v=pltpu.VMEM((B, D), jnp.float32),
            zbuf=pltpu.VMEM((16, D), jnp.float32),
            sem=pltpu.SemaphoreType.DMA,
        ),
    )
    def seg_sum(x_hbm, src_hbm, dst_hbm, out_hbm, acc, src_v, dst_v,
                rows_v, zbuf, sem):
        c = lax.axis_index("c")
        s = lax.axis_index("s")
        wid = c * NS + s

        z = jnp.zeros((16,), jnp.float32)
        for i in range(16):
            for j in range(D // 16):
                zbuf[i, pl.ds(j * 16, 16)] = z

        def zero_body(k, carry):
            pltpu.sync_copy(zbuf, acc.at[pl.ds(s * RPT + k * 16, 16)])
            return carry
        lax.fori_loop(0, RPT // 16, zero_body, 0)

        pltpu.sync_copy(src_hbm.at[wid], src_v)
        pltpu.sync_copy(dst_hbm.at[wid], dst_v)

        plsc.subcore_barrier()

        def edge_body(j, carry):
            pltpu.async_copy(x_hbm.at[src_v.at[j]], rows_v, sem).wait()
            pltpu.sync_copy(rows_v, acc.at[dst_v.at[j]], add=True)
            return carry
        lax.fori_loop(0, NBLK, edge_body, 0)

        plsc.subcore_barrier()

        pltpu.sync_copy(acc.at[pl.ds(s * RPT, RPT)],
                        out_hbm.at[pl.ds(c * N_PAD + s * RPT, RPT)])

    return seg_sum(x, src3, dst3)


def _tc_linear(p0, p1, W):
    BN = 1000

    def body(p0_ref, p1_ref, w_ref, o_ref):
        agg = p0_ref[...] + p1_ref[...]
        o_ref[...] = lax.dot_general(
            agg, w_ref[...], (((1,), (1,)), ((), ())),
            preferred_element_type=jnp.float32)

    return pl.pallas_call(
        body,
        grid=(N // BN,),
        in_specs=[
            pl.BlockSpec((BN, D), lambda i: (i, 0)),
            pl.BlockSpec((BN, D), lambda i: (i, 0)),
            pl.BlockSpec((D, D), lambda i: (0, 0)),
        ],
        out_specs=pl.BlockSpec((BN, D), lambda i: (i, 0)),
        out_shape=jax.ShapeDtypeStruct((N, D), jnp.float32),
    )(p0, p1, W)


def kernel(x, edge_index, W):
    dst = edge_index[0]
    src = edge_index[1]
    pad = E_PAD - E
    src_p = jnp.concatenate([src, jnp.zeros((pad,), jnp.int32)])
    dst_p = jnp.concatenate([dst, jnp.full((pad,), N, jnp.int32)])
    src3 = src_p.reshape(NW, NBLK, B)
    dst3 = dst_p.reshape(NW, NBLK, B)

    partials = _sc_segment_sum(x, src3, dst3)
    p0 = partials[:N]
    p1 = partials[N_PAD:N_PAD + N]
    return _tc_linear(p0, p1, W)
